# baseline (speedup 1.0000x reference)
import functools

import jax
import jax.numpy as jnp
from jax import lax
from jax.experimental import pallas as pl
from jax.experimental.pallas import tpu as pltpu
from jax.experimental.pallas import tpu_sc as plsc

_NUM_BINS = 1000000
_B = 16384
_NC = 2
_NS = 16
_NW = _NC * _NS
_PER_W = _B // _NW
_L = 16


def _mix(h):
    h = h ^ (h >> jnp.uint32(16))
    h = h * jnp.uint32(0x85EBCA6B)
    h = h ^ (h >> jnp.uint32(13))
    h = h * jnp.uint32(0xC2B2AE35)
    h = h ^ (h >> jnp.uint32(16))
    return h


def _body(f1_hbm, f2_hbm, out_hbm, f1_v, f2_v, o_v):
    wid = lax.axis_index("s") * _NC + lax.axis_index("c")
    base = wid * _PER_W
    pltpu.sync_copy(f1_hbm.at[pl.ds(base, _PER_W)], f1_v)
    pltpu.sync_copy(f2_hbm.at[pl.ds(base, _PER_W)], f2_v)
    for i in range(_PER_W // _L):
        a = f1_v[pl.ds(i * _L, _L)].astype(jnp.uint32)
        b = f2_v[pl.ds(i * _L, _L)].astype(jnp.uint32)
        h = _mix(a)
        h = h ^ (_mix(b) + jnp.uint32(0x9E3779B9)
                 + (h << jnp.uint32(6)) + (h >> jnp.uint32(2)))
        h = _mix(h)
        o_v[pl.ds(i * _L, _L)] = (h % jnp.uint32(_NUM_BINS)).astype(jnp.int32)
    pltpu.sync_copy(o_v, out_hbm.at[pl.ds(base, _PER_W)])


@jax.jit
def kernel(feat1, feat2):
    mesh = plsc.VectorSubcoreMesh(core_axis_name="c", subcore_axis_name="s")
    f = pl.kernel(
        _body,
        mesh=mesh,
        out_type=jax.ShapeDtypeStruct((_B,), jnp.int32),
        scratch_types=[
            pltpu.VMEM((_PER_W,), jnp.int32),
            pltpu.VMEM((_PER_W,), jnp.int32),
            pltpu.VMEM((_PER_W,), jnp.int32),
        ],
    )
    return f(feat1, feat2)


# speedup vs baseline: 1.0423x; 1.0423x over previous
import jax
import jax.numpy as jnp
from jax import lax
from jax.experimental import pallas as pl
from jax.experimental.pallas import tpu as pltpu
from jax.experimental.pallas import tpu_sc as plsc

_NUM_BINS = 1000000
_B = 16384
_NC = 1
_NS = 16
_NW = _NC * _NS
_PER_W = _B // _NW
_L = 16


def _mix(h):
    h = h ^ (h >> jnp.uint32(16))
    h = h * jnp.uint32(0x85EBCA6B)
    h = h ^ (h >> jnp.uint32(13))
    h = h * jnp.uint32(0xC2B2AE35)
    h = h ^ (h >> jnp.uint32(16))
    return h


def _body(f1_hbm, f2_hbm, out_hbm, f1_v, f2_v, o_v, sem):
    wid = lax.axis_index("s") * _NC + lax.axis_index("c")
    base = wid * _PER_W
    cp1 = pltpu.async_copy(f1_hbm.at[pl.ds(base, _PER_W)], f1_v, sem)
    cp2 = pltpu.async_copy(f2_hbm.at[pl.ds(base, _PER_W)], f2_v, sem)
    cp1.wait()
    cp2.wait()
    for i in range(_PER_W // _L):
        a = f1_v[pl.ds(i * _L, _L)].astype(jnp.uint32)
        b = f2_v[pl.ds(i * _L, _L)].astype(jnp.uint32)
        h = _mix(a)
        h = h ^ (_mix(b) + jnp.uint32(0x9E3779B9)
                 + (h << jnp.uint32(6)) + (h >> jnp.uint32(2)))
        h = _mix(h)
        o_v[pl.ds(i * _L, _L)] = (h % jnp.uint32(_NUM_BINS)).astype(jnp.int32)
    pltpu.sync_copy(o_v, out_hbm.at[pl.ds(base, _PER_W)])


@jax.jit
def kernel(feat1, feat2):
    mesh = plsc.VectorSubcoreMesh(
        core_axis_name="c", subcore_axis_name="s", num_cores=_NC)
    f = pl.kernel(
        _body,
        mesh=mesh,
        out_type=jax.ShapeDtypeStruct((_B,), jnp.int32),
        scratch_types=[
            pltpu.VMEM((_PER_W,), jnp.int32),
            pltpu.VMEM((_PER_W,), jnp.int32),
            pltpu.VMEM((_PER_W,), jnp.int32),
            pltpu.SemaphoreType.DMA,
        ],
    )
    return f(feat1, feat2)
